# baseline (speedup 1.0000x reference)
import functools

import jax
import jax.numpy as jnp
from jax import lax
from jax.experimental import pallas as pl
from jax.experimental.pallas import tpu as pltpu
from jax.experimental.pallas import tpu_sc as plsc

N = 2048
D = 1024
H = 2048
E = 8
K = 2
A = N * K
BLK = 256
NBLK = 24
PADMAX = NBLK * BLK

NC = 2
NS = 16
NW = NC * NS
CHUNK = A // NW
VPC = CHUNK // 16
ROWS = 32
RCH = CHUNK // ROWS

TOK_C = N // NW
TOK_SUB = 16


def _gating_body(x_ref, wg_ref, bg_ref, wn_ref, bn_ref, noise_ref,
                 eid_ref, gw_ref):
    xv = x_ref[...]
    zg = lax.dot_general(xv, wg_ref[...], (((1,), (1,)), ((), ())),
                         preferred_element_type=jnp.float32) + bg_ref[...]
    zn = lax.dot_general(xv, wn_ref[...], (((1,), (1,)), ((), ())),
                         preferred_element_type=jnp.float32) + bn_ref[...]
    sp = jnp.maximum(zn, 0.0) + jnp.log1p(jnp.exp(-jnp.abs(zn)))
    noisy = zg + noise_ref[...] * sp
    nt = noisy.T
    lane = lax.broadcasted_iota(jnp.int32, (E, N), 0)
    v1 = jnp.max(nt, axis=0, keepdims=True)
    i1 = jnp.min(jnp.where(nt == v1, lane, E), axis=0, keepdims=True)
    masked = jnp.where(lane == i1, -jnp.inf, nt)
    v2 = jnp.max(masked, axis=0, keepdims=True)
    i2 = jnp.min(jnp.where(masked == v2, lane, E), axis=0, keepdims=True)
    eid_ref[...] = jnp.concatenate([i1, i2], axis=0)
    lanec = lax.broadcasted_iota(jnp.int32, (N, E), 1)
    v1c = jnp.max(noisy, axis=1, keepdims=True)
    i1c = jnp.min(jnp.where(noisy == v1c, lanec, E), axis=1, keepdims=True)
    v2c = jnp.max(jnp.where(lanec == i1c, -jnp.inf, noisy), axis=1,
                  keepdims=True)
    g1c = 1.0 / (1.0 + jnp.exp(v2c - v1c))
    gw_ref[...] = jnp.concatenate(
        [jnp.broadcast_to(g1c, (N, 16)),
         jnp.broadcast_to(1.0 - g1c, (N, 16))], axis=0)


def _gating(x, wg, bg, wn, bn, noise):
    return pl.pallas_call(
        _gating_body,
        out_shape=(
            jax.ShapeDtypeStruct((K, N), jnp.int32),
            jax.ShapeDtypeStruct((K * N, 16), jnp.float32),
        ),
    )(x, wg, bg, wn, bn, noise)


def _routing_body(eid_hbm, x_hbm,
                  pos_hbm, xs_hbm, be_hbm, meta_hbm,
                  eid_v, pos_v, cnt_v, bef_v,
                  row0_v, row1_v, row2_v, row3_v, be_v, meta_v,
                  sg0, sg1, sg2, sg3, ss0, ss1, ss2, ss3):
    w = lax.axis_index("s") * NC + lax.axis_index("c")
    lane = lax.broadcasted_iota(jnp.int32, (16,), 0)
    pltpu.sync_copy(eid_hbm, eid_v)

    my0 = w * VPC
    cnt_v[...] = jnp.zeros((16,), jnp.int32)
    bef_v[...] = jnp.zeros((16,), jnp.int32)
    ones = jnp.ones((16,), jnp.int32)

    def tot_step(jv, _):
        plsc.addupdate_scatter(cnt_v, [eid_v[pl.ds(jv * 16, 16)]], ones)
        return _

    def bef_step(jv, _):
        plsc.addupdate_scatter(bef_v, [eid_v[pl.ds(jv * 16, 16)]], ones)
        return _

    lax.fori_loop(0, A // 16, tot_step, 0, unroll=4)
    lax.fori_loop(0, my0, bef_step, 0)
    total = cnt_v[...]
    before = bef_v[...]

    padded = ((total + (BLK - 1)) >> 8) << 8
    off = plsc.cumsum(padded) - padded
    run = off + before

    posl = []
    for j in range(VPC):
        v = eid_v[pl.ds((my0 + j) * 16, 16)]
        posv = jnp.zeros((16,), jnp.int32)
        for e in range(E):
            m = v == e
            mi = m.astype(jnp.int32)
            excl = plsc.cumsum(mi) - mi
            run_e = jnp.sum(jnp.where(lane == e, run, 0))
            posv = jnp.where(m, run_e + excl, posv)
            p = jnp.sum(mi)
            run = jnp.where(lane == e, run + p, run)
        pos_v[pl.ds(j * 16, 16)] = posv
        posl.append(posv)
    pltpu.sync_copy(pos_v, pos_hbm.at[pl.ds(w * CHUNK, CHUNK)])

    rows = (row0_v, row1_v, row2_v, row3_v)
    gsem = (sg0, sg1, sg2, sg3)
    ssem = (ss0, ss1, ss2, ss3)

    def gath(j):
        tokv = (w * CHUNK + j * 16 + lane) & (N - 1)
        return pltpu.async_copy(x_hbm.at[tokv], rows[j % 4], gsem[j % 4])

    def scat(j):
        return pltpu.async_copy(rows[j % 4], xs_hbm.at[posl[j]], ssem[j % 4])

    gh = {j: gath(j) for j in range(min(3, VPC))}
    sh = {}
    for j in range(VPC):
        if j + 3 < VPC:
            if j - 1 >= 0:
                sh[j - 1].wait()
            gh[j + 3] = gath(j + 3)
        gh[j].wait()
        sh[j] = scat(j)
    for j in range(max(0, VPC - 4), VPC):
        if j not in sh:
            continue
        sh[j].wait()

    @pl.when(w == 0)
    def _():
        nact = jnp.sum(padded) >> 8
        nonempty = jnp.where(total > 0, lane, 0)
        edef = jnp.max(nonempty)
        for j2 in range(NBLK // 16 + 1):
            start = (j2 * 16 + lane) * BLK
            eb = jnp.zeros((16,), jnp.int32) + edef
            for e in range(E):
                off_e = jnp.sum(jnp.where(lane == e, off, 0))
                pad_e = jnp.sum(jnp.where(lane == e, padded, 0))
                m2 = (start >= off_e) & (start < off_e + pad_e)
                eb = jnp.where(m2, e, eb)
            be_v[j2, :] = eb
        meta_v[...] = jnp.zeros((16,), jnp.int32) + nact
        pltpu.sync_copy(be_v, be_hbm)
        pltpu.sync_copy(meta_v, meta_hbm)


def _routing(eid_flat, x):
    mesh = plsc.VectorSubcoreMesh(core_axis_name="c", subcore_axis_name="s")
    f = functools.partial(
        pl.kernel,
        mesh=mesh,
        compiler_params=pltpu.CompilerParams(needs_layout_passes=False),
        out_type=(
            jax.ShapeDtypeStruct((A,), jnp.int32),
            jax.ShapeDtypeStruct((PADMAX, D), jnp.float32),
            jax.ShapeDtypeStruct((2, 16), jnp.int32),
            jax.ShapeDtypeStruct((16,), jnp.int32),
        ),
        scratch_types=[
            pltpu.VMEM((A,), jnp.int32),
            pltpu.VMEM((CHUNK,), jnp.int32),
            pltpu.VMEM((16,), jnp.int32),
            pltpu.VMEM((16,), jnp.int32),
            pltpu.VMEM((16, D), jnp.float32),
            pltpu.VMEM((16, D), jnp.float32),
            pltpu.VMEM((16, D), jnp.float32),
            pltpu.VMEM((16, D), jnp.float32),
            pltpu.VMEM((2, 16), jnp.int32),
            pltpu.VMEM((16,), jnp.int32),
            pltpu.SemaphoreType.DMA,
            pltpu.SemaphoreType.DMA,
            pltpu.SemaphoreType.DMA,
            pltpu.SemaphoreType.DMA,
            pltpu.SemaphoreType.DMA,
            pltpu.SemaphoreType.DMA,
            pltpu.SemaphoreType.DMA,
            pltpu.SemaphoreType.DMA,
        ],
    )(_routing_body)
    return f(eid_flat, x)


def _ffn_body(be_ref, meta_ref, xs_ref, w1_ref, b1_ref, w2_ref, b2_ref,
              ys_ref):
    i = pl.program_id(0)

    @pl.when(i < meta_ref[0])
    def _():
        xb = xs_ref[...]
        h = lax.dot_general(xb, w1_ref[0],
                            (((1,), (1,)), ((), ())),
                            preferred_element_type=jnp.float32)
        h = jnp.maximum(h + b1_ref[0], 0.0)
        o = lax.dot_general(h, w2_ref[0],
                            (((1,), (1,)), ((), ())),
                            preferred_element_type=jnp.float32)
        ys_ref[...] = o + b2_ref[0]


def _ffn(be32, meta, xs, W1, b1, W2, b2):
    grid_spec = pltpu.PrefetchScalarGridSpec(
        num_scalar_prefetch=2,
        grid=(NBLK,),
        in_specs=[
            pl.BlockSpec((BLK, D), lambda i, be, m: (i, 0)),
            pl.BlockSpec((1, H, D), lambda i, be, m: (be[i], 0, 0)),
            pl.BlockSpec((1, 1, H), lambda i, be, m: (be[i], 0, 0)),
            pl.BlockSpec((1, D, H), lambda i, be, m: (be[i], 0, 0)),
            pl.BlockSpec((1, 1, D), lambda i, be, m: (be[i], 0, 0)),
        ],
        out_specs=pl.BlockSpec((BLK, D), lambda i, be, m: (i, 0)),
    )
    return pl.pallas_call(
        _ffn_body,
        grid_spec=grid_spec,
        out_shape=jax.ShapeDtypeStruct((PADMAX, D), jnp.float32),
    )(be32, meta, xs, W1, b1, W2, b2)


def _combine_body(ys_hbm, pos_hbm, gw_hbm, out_hbm,
                  p1_v, p2_v, g1_v, g2_v,
                  ra0, ra1, rb0, rb1, ob0, ob1,
                  sa0, sa1, sb0, sb1, so0, so1):
    w = lax.axis_index("s") * NC + lax.axis_index("c")
    t0 = w * TOK_C
    pltpu.sync_copy(pos_hbm.at[pl.ds(t0, TOK_C)], p1_v)
    pltpu.sync_copy(pos_hbm.at[pl.ds(N + t0, TOK_C)], p2_v)
    pltpu.sync_copy(gw_hbm.at[pl.ds(t0, TOK_C)], g1_v)
    pltpu.sync_copy(gw_hbm.at[pl.ds(N + t0, TOK_C)], g2_v)
    bufs = ((ra0, rb0, ob0, sa0, sb0, so0),
            (ra1, rb1, ob1, sa1, sb1, so1))
    nsub = TOK_C // TOK_SUB

    def issue(c):
        ra, rb, _, sa, sb, _2 = bufs[c % 2]
        h1 = pltpu.async_copy(
            ys_hbm.at[p1_v[pl.ds(c * TOK_SUB, TOK_SUB)]], ra, sa)
        h2 = pltpu.async_copy(
            ys_hbm.at[p2_v[pl.ds(c * TOK_SUB, TOK_SUB)]], rb, sb)
        return h1, h2

    hs = {0: issue(0)}
    oh = {}
    for c in range(nsub):
        if c + 1 < nsub:
            hs[c + 1] = issue(c + 1)
        hs[c][0].wait()
        hs[c][1].wait()
        ra, rb, ob, _, _2, so = bufs[c % 2]
        if c - 2 >= 0:
            oh[c - 2].wait()
        for g in range(TOK_SUB // 8):
            g1r = [g1_v[c * TOK_SUB + g * 8 + r, :] for r in range(8)]
            g2r = [g2_v[c * TOK_SUB + g * 8 + r, :] for r in range(8)]

            def col_step(cc, _u, g=g, g1r=g1r, g2r=g2r,
                         ra=ra, rb=rb, ob=ob):
                s = pl.ds(cc * 16, 16)
                for r in range(8):
                    rr = g * 8 + r
                    ob[rr, s] = g1r[r] * ra[rr, s] + g2r[r] * rb[rr, s]
                return _u

            lax.fori_loop(0, D // 16, col_step, 0, unroll=2)
        oh[c] = pltpu.async_copy(
            ob, out_hbm.at[pl.ds(t0 + c * TOK_SUB, TOK_SUB)], so)
    oh[nsub - 2].wait()
    oh[nsub - 1].wait()


def _combine(ys, pos, gw):
    mesh = plsc.VectorSubcoreMesh(core_axis_name="c", subcore_axis_name="s")
    f = functools.partial(
        pl.kernel,
        mesh=mesh,
        compiler_params=pltpu.CompilerParams(needs_layout_passes=False),
        out_type=jax.ShapeDtypeStruct((N, D), jnp.float32),
        scratch_types=[
            pltpu.VMEM((TOK_C,), jnp.int32),
            pltpu.VMEM((TOK_C,), jnp.int32),
            pltpu.VMEM((TOK_C, 16), jnp.float32),
            pltpu.VMEM((TOK_C, 16), jnp.float32),
        ] + [pltpu.VMEM((TOK_SUB, D), jnp.float32)] * 6
          + [pltpu.SemaphoreType.DMA] * 6,
    )(_combine_body)
    return f(ys, pos, gw)


def kernel(x, Wg_w, Wg_b, Wn_w, Wn_b, W1, b1, W2, b2, noise):
    eid, gw = _gating(x, Wg_w, Wg_b.reshape(1, E), Wn_w, Wn_b.reshape(1, E),
                      noise)
    pos, xs, be, meta, = _routing(eid.reshape(A), x)
    ys = _ffn(be.reshape(2 * 16), meta, xs, W1, b1.reshape(E, 1, H), W2,
              b2.reshape(E, 1, D))
    return _combine(ys, pos, gw)


# speedup vs baseline: 1.0235x; 1.0235x over previous
import functools

import jax
import jax.numpy as jnp
from jax import lax
from jax.experimental import pallas as pl
from jax.experimental.pallas import tpu as pltpu
from jax.experimental.pallas import tpu_sc as plsc

N = 2048
D = 1024
H = 2048
E = 8
K = 2
A = N * K
BLK = 256
NBLK = 24
PADMAX = NBLK * BLK

NC = 2
NS = 16
NW = NC * NS
CHUNK = A // NW
VPC = CHUNK // 16
ROWS = 32
RCH = CHUNK // ROWS

TOK_C = N // NW
TOK_SUB = 16


def _gating_body(x_ref, wg_ref, bg_ref, wn_ref, bn_ref, noise_ref,
                 eid_ref, gw_ref):
    xv = x_ref[...]
    zg = lax.dot_general(xv, wg_ref[...], (((1,), (1,)), ((), ())),
                         preferred_element_type=jnp.float32) + bg_ref[...]
    zn = lax.dot_general(xv, wn_ref[...], (((1,), (1,)), ((), ())),
                         preferred_element_type=jnp.float32) + bn_ref[...]
    sp = jnp.maximum(zn, 0.0) + jnp.log1p(jnp.exp(-jnp.abs(zn)))
    noisy = zg + noise_ref[...] * sp
    nt = noisy.T
    lane = lax.broadcasted_iota(jnp.int32, (E, N), 0)
    v1 = jnp.max(nt, axis=0, keepdims=True)
    i1 = jnp.min(jnp.where(nt == v1, lane, E), axis=0, keepdims=True)
    masked = jnp.where(lane == i1, -jnp.inf, nt)
    v2 = jnp.max(masked, axis=0, keepdims=True)
    i2 = jnp.min(jnp.where(masked == v2, lane, E), axis=0, keepdims=True)
    eid_ref[...] = jnp.concatenate([i1, i2], axis=0)
    lanec = lax.broadcasted_iota(jnp.int32, (N, E), 1)
    v1c = jnp.max(noisy, axis=1, keepdims=True)
    i1c = jnp.min(jnp.where(noisy == v1c, lanec, E), axis=1, keepdims=True)
    v2c = jnp.max(jnp.where(lanec == i1c, -jnp.inf, noisy), axis=1,
                  keepdims=True)
    g1c = 1.0 / (1.0 + jnp.exp(v2c - v1c))
    gw_ref[...] = jnp.concatenate(
        [jnp.broadcast_to(g1c, (N, 16)),
         jnp.broadcast_to(1.0 - g1c, (N, 16))], axis=0)


def _gating(x, wg, bg, wn, bn, noise):
    return pl.pallas_call(
        _gating_body,
        out_shape=(
            jax.ShapeDtypeStruct((K, N), jnp.int32),
            jax.ShapeDtypeStruct((K * N, 16), jnp.float32),
        ),
    )(x, wg, bg, wn, bn, noise)


def _routing_body(eid_hbm, x_hbm,
                  pos_hbm, xs_hbm, be_hbm, meta_hbm,
                  eid_v, pos_v, cnt_v, bef_v,
                  row0_v, row1_v, row2_v, row3_v, be_v, meta_v,
                  sg0, sg1, sg2, sg3, ss0, ss1, ss2, ss3):
    w = lax.axis_index("s") * NC + lax.axis_index("c")
    lane = lax.broadcasted_iota(jnp.int32, (16,), 0)
    pltpu.sync_copy(eid_hbm, eid_v)

    my0 = w * VPC
    cnt_v[...] = jnp.zeros((16,), jnp.int32)
    bef_v[...] = jnp.zeros((16,), jnp.int32)
    ones = jnp.ones((16,), jnp.int32)

    def tot_step(jv, _):
        plsc.addupdate_scatter(cnt_v, [eid_v[pl.ds(jv * 16, 16)]], ones)
        return _

    def bef_step(jv, _):
        plsc.addupdate_scatter(bef_v, [eid_v[pl.ds(jv * 16, 16)]], ones)
        return _

    lax.fori_loop(0, A // 16, tot_step, 0)
    lax.fori_loop(0, my0, bef_step, 0)
    total = cnt_v[...]
    before = bef_v[...]

    padded = ((total + (BLK - 1)) >> 8) << 8
    off = plsc.cumsum(padded) - padded
    run = off + before

    posl = []
    for j in range(VPC):
        v = eid_v[pl.ds((my0 + j) * 16, 16)]
        posv = jnp.zeros((16,), jnp.int32)
        for e in range(E):
            m = v == e
            mi = m.astype(jnp.int32)
            excl = plsc.cumsum(mi) - mi
            run_e = jnp.sum(jnp.where(lane == e, run, 0))
            posv = jnp.where(m, run_e + excl, posv)
            p = jnp.sum(mi)
            run = jnp.where(lane == e, run + p, run)
        pos_v[pl.ds(j * 16, 16)] = posv
        posl.append(posv)
    pltpu.sync_copy(pos_v, pos_hbm.at[pl.ds(w * CHUNK, CHUNK)])

    rows = (row0_v, row1_v, row2_v, row3_v)
    gsem = (sg0, sg1, sg2, sg3)
    ssem = (ss0, ss1, ss2, ss3)

    def gath(j):
        tokv = (w * CHUNK + j * 16 + lane) & (N - 1)
        return pltpu.async_copy(x_hbm.at[tokv], rows[j % 4], gsem[j % 4])

    def scat(j):
        return pltpu.async_copy(rows[j % 4], xs_hbm.at[posl[j]], ssem[j % 4])

    gh = {j: gath(j) for j in range(min(3, VPC))}
    sh = {}
    for j in range(VPC):
        if j + 3 < VPC:
            if j - 1 >= 0:
                sh[j - 1].wait()
            gh[j + 3] = gath(j + 3)
        gh[j].wait()
        sh[j] = scat(j)
    for j in range(max(0, VPC - 4), VPC):
        if j not in sh:
            continue
        sh[j].wait()

    @pl.when(w == 0)
    def _():
        nact = jnp.sum(padded) >> 8
        nonempty = jnp.where(total > 0, lane, 0)
        edef = jnp.max(nonempty)
        for j2 in range(NBLK // 16 + 1):
            start = (j2 * 16 + lane) * BLK
            eb = jnp.zeros((16,), jnp.int32) + edef
            for e in range(E):
                off_e = jnp.sum(jnp.where(lane == e, off, 0))
                pad_e = jnp.sum(jnp.where(lane == e, padded, 0))
                m2 = (start >= off_e) & (start < off_e + pad_e)
                eb = jnp.where(m2, e, eb)
            be_v[j2, :] = eb
        meta_v[...] = jnp.zeros((16,), jnp.int32) + nact
        pltpu.sync_copy(be_v, be_hbm)
        pltpu.sync_copy(meta_v, meta_hbm)


def _routing(eid_flat, x):
    mesh = plsc.VectorSubcoreMesh(core_axis_name="c", subcore_axis_name="s")
    f = functools.partial(
        pl.kernel,
        mesh=mesh,
        compiler_params=pltpu.CompilerParams(needs_layout_passes=False),
        out_type=(
            jax.ShapeDtypeStruct((A,), jnp.int32),
            jax.ShapeDtypeStruct((PADMAX, D), jnp.float32),
            jax.ShapeDtypeStruct((2, 16), jnp.int32),
            jax.ShapeDtypeStruct((16,), jnp.int32),
        ),
        scratch_types=[
            pltpu.VMEM((A,), jnp.int32),
            pltpu.VMEM((CHUNK,), jnp.int32),
            pltpu.VMEM((16,), jnp.int32),
            pltpu.VMEM((16,), jnp.int32),
            pltpu.VMEM((16, D), jnp.float32),
            pltpu.VMEM((16, D), jnp.float32),
            pltpu.VMEM((16, D), jnp.float32),
            pltpu.VMEM((16, D), jnp.float32),
            pltpu.VMEM((2, 16), jnp.int32),
            pltpu.VMEM((16,), jnp.int32),
            pltpu.SemaphoreType.DMA,
            pltpu.SemaphoreType.DMA,
            pltpu.SemaphoreType.DMA,
            pltpu.SemaphoreType.DMA,
            pltpu.SemaphoreType.DMA,
            pltpu.SemaphoreType.DMA,
            pltpu.SemaphoreType.DMA,
            pltpu.SemaphoreType.DMA,
        ],
    )(_routing_body)
    return f(eid_flat, x)


def _ffn_body(be_ref, meta_ref, xs_ref, w1_ref, b1_ref, w2_ref, b2_ref,
              ys_ref):
    i = pl.program_id(0)

    @pl.when(i < meta_ref[0])
    def _():
        xb = xs_ref[...]
        h = lax.dot_general(xb, w1_ref[0],
                            (((1,), (1,)), ((), ())),
                            preferred_element_type=jnp.float32)
        h = jnp.maximum(h + b1_ref[0], 0.0)
        o = lax.dot_general(h, w2_ref[0],
                            (((1,), (1,)), ((), ())),
                            preferred_element_type=jnp.float32)
        ys_ref[...] = o + b2_ref[0]


def _ffn(be32, meta, xs, W1, b1, W2, b2):
    grid_spec = pltpu.PrefetchScalarGridSpec(
        num_scalar_prefetch=2,
        grid=(NBLK,),
        in_specs=[
            pl.BlockSpec((BLK, D),
                         lambda i, be, m: (jnp.minimum(i, m[0] - 1), 0)),
            pl.BlockSpec((1, H, D), lambda i, be, m: (be[i], 0, 0)),
            pl.BlockSpec((1, 1, H), lambda i, be, m: (be[i], 0, 0)),
            pl.BlockSpec((1, D, H), lambda i, be, m: (be[i], 0, 0)),
            pl.BlockSpec((1, 1, D), lambda i, be, m: (be[i], 0, 0)),
        ],
        out_specs=pl.BlockSpec((BLK, D),
                               lambda i, be, m: (jnp.minimum(i, m[0] - 1), 0)),
    )
    return pl.pallas_call(
        _ffn_body,
        grid_spec=grid_spec,
        out_shape=jax.ShapeDtypeStruct((PADMAX, D), jnp.float32),
    )(be32, meta, xs, W1, b1, W2, b2)


def _combine_body(ys_hbm, pos_hbm, gw_hbm, out_hbm,
                  p1_v, p2_v, g1_v, g2_v,
                  ra0, ra1, rb0, rb1, ob0, ob1,
                  sa0, sa1, sb0, sb1, so0, so1):
    w = lax.axis_index("s") * NC + lax.axis_index("c")
    t0 = w * TOK_C
    pltpu.sync_copy(pos_hbm.at[pl.ds(t0, TOK_C)], p1_v)
    pltpu.sync_copy(pos_hbm.at[pl.ds(N + t0, TOK_C)], p2_v)
    pltpu.sync_copy(gw_hbm.at[pl.ds(t0, TOK_C)], g1_v)
    pltpu.sync_copy(gw_hbm.at[pl.ds(N + t0, TOK_C)], g2_v)
    bufs = ((ra0, rb0, ob0, sa0, sb0, so0),
            (ra1, rb1, ob1, sa1, sb1, so1))
    nsub = TOK_C // TOK_SUB

    def issue(c):
        ra, rb, _, sa, sb, _2 = bufs[c % 2]
        h1 = pltpu.async_copy(
            ys_hbm.at[p1_v[pl.ds(c * TOK_SUB, TOK_SUB)]], ra, sa)
        h2 = pltpu.async_copy(
            ys_hbm.at[p2_v[pl.ds(c * TOK_SUB, TOK_SUB)]], rb, sb)
        return h1, h2

    hs = {0: issue(0)}
    oh = {}
    for c in range(nsub):
        if c + 1 < nsub:
            hs[c + 1] = issue(c + 1)
        hs[c][0].wait()
        hs[c][1].wait()
        ra, rb, ob, _, _2, so = bufs[c % 2]
        if c - 2 >= 0:
            oh[c - 2].wait()
        for g in range(TOK_SUB // 8):
            g1r = [g1_v[c * TOK_SUB + g * 8 + r, :] for r in range(8)]
            g2r = [g2_v[c * TOK_SUB + g * 8 + r, :] for r in range(8)]

            def col_step(cc, _u, g=g, g1r=g1r, g2r=g2r,
                         ra=ra, rb=rb, ob=ob):
                s = pl.ds(cc * 16, 16)
                for r in range(8):
                    rr = g * 8 + r
                    ob[rr, s] = g1r[r] * ra[rr, s] + g2r[r] * rb[rr, s]
                return _u

            lax.fori_loop(0, D // 16, col_step, 0)
        oh[c] = pltpu.async_copy(
            ob, out_hbm.at[pl.ds(t0 + c * TOK_SUB, TOK_SUB)], so)
    oh[nsub - 2].wait()
    oh[nsub - 1].wait()


def _combine(ys, pos, gw):
    mesh = plsc.VectorSubcoreMesh(core_axis_name="c", subcore_axis_name="s")
    f = functools.partial(
        pl.kernel,
        mesh=mesh,
        compiler_params=pltpu.CompilerParams(needs_layout_passes=False),
        out_type=jax.ShapeDtypeStruct((N, D), jnp.float32),
        scratch_types=[
            pltpu.VMEM((TOK_C,), jnp.int32),
            pltpu.VMEM((TOK_C,), jnp.int32),
            pltpu.VMEM((TOK_C, 16), jnp.float32),
            pltpu.VMEM((TOK_C, 16), jnp.float32),
        ] + [pltpu.VMEM((TOK_SUB, D), jnp.float32)] * 6
          + [pltpu.SemaphoreType.DMA] * 6,
    )(_combine_body)
    return f(ys, pos, gw)


def kernel(x, Wg_w, Wg_b, Wn_w, Wn_b, W1, b1, W2, b2, noise):
    eid, gw = _gating(x, Wg_w, Wg_b.reshape(1, E), Wn_w, Wn_b.reshape(1, E),
                      noise)
    pos, xs, be, meta, = _routing(eid.reshape(A), x)
    ys = _ffn(be.reshape(2 * 16), meta, xs, W1, b1.reshape(E, 1, H), W2,
              b2.reshape(E, 1, D))
    return _combine(ys, pos, gw)
